# baseline (speedup 1.0000x reference)
import functools

import jax
import jax.numpy as jnp
from jax import lax
from jax.experimental import pallas as pl
from jax.experimental.pallas import tpu as pltpu
from jax.experimental.pallas import tpu_sc as plsc

NUM_NODES = 1000000
EMBED_DIM = 32
BATCH = 16384
HIST = 50

_B_TOTAL = BATCH * HIST
_NC, _NS = 2, 16
_NW = _NC * _NS
_CHUNK = 128
_PER_W = _B_TOTAL // _NW
_NCHUNK = _PER_W // _CHUNK
_K = 10
_GROUP = _K * _CHUNK
_NG = _NCHUNK // _K


def _gather_body(idx_hbm, table_hbm, out_hbm, idx_v, buf0, buf1,
                 gsem0, gsem1, osem0, osem1):
    wid = lax.axis_index("s") * _NC + lax.axis_index("c")
    chunk_base = wid * _NCHUNK
    row_base = wid * _PER_W

    pltpu.sync_copy(idx_hbm.at[pl.ds(row_base, _PER_W)], idx_v)

    def fire(g, buf, gsem):
        pltpu.async_copy(
            table_hbm.at[idx_v.at[pl.ds(g * _GROUP, _GROUP)]],
            buf,
            gsem)

    def drain_gathers(buf, gsem):
        pltpu.make_async_copy(out_hbm.at[pl.ds(0, _GROUP)], buf, gsem).wait()

    def start_out(g, buf, osem):
        pltpu.async_copy(buf, out_hbm.at[pl.ds(row_base + g * _GROUP, _GROUP)],
                         osem)

    def wait_out(buf, osem):
        pltpu.make_async_copy(out_hbm.at[pl.ds(0, _GROUP)], buf, osem).wait()

    fire(0, buf0, gsem0)

    def outer(t, _):
        g0 = 2 * t
        g1 = 2 * t + 1
        drain_gathers(buf0, gsem0)

        @pl.when(t > 0)
        def _():
            wait_out(buf1, osem1)
        fire(g1, buf1, gsem1)
        start_out(g0, buf0, osem0)

        drain_gathers(buf1, gsem1)

        @pl.when(t < _NG // 2 - 1)
        def _():
            wait_out(buf0, osem0)
            fire(g0 + 2, buf0, gsem0)
        start_out(g1, buf1, osem1)
        return 0

    lax.fori_loop(0, _NG // 2, outer, 0)
    wait_out(buf0, osem0)
    wait_out(buf1, osem1)


_sc_gather = pl.kernel(
    _gather_body,
    mesh=plsc.VectorSubcoreMesh(core_axis_name="c", subcore_axis_name="s"),
    out_type=jax.ShapeDtypeStruct((_B_TOTAL, EMBED_DIM), jnp.float32),
    scratch_types=[
        pltpu.VMEM((_PER_W,), jnp.int32),
        pltpu.VMEM((_GROUP, EMBED_DIM), jnp.float32),
        pltpu.VMEM((_GROUP, EMBED_DIM), jnp.float32),
        pltpu.SemaphoreType.DMA,
        pltpu.SemaphoreType.DMA,
        pltpu.SemaphoreType.DMA,
        pltpu.SemaphoreType.DMA,
    ],
    compiler_params=pltpu.CompilerParams(use_tc_tiling_on_sc=False),
)


def kernel(indices, embeddings):
    flat = _sc_gather(indices.reshape(_B_TOTAL), embeddings)
    return flat.reshape(BATCH, HIST, EMBED_DIM)


# speedup vs baseline: 1.6570x; 1.6570x over previous
import functools

import jax
import jax.numpy as jnp
from jax import lax
from jax.experimental import pallas as pl
from jax.experimental.pallas import tpu as pltpu
from jax.experimental.pallas import tpu_sc as plsc

NUM_NODES = 1000000
EMBED_DIM = 32
BATCH = 16384
HIST = 50

_NC, _NS = 2, 16
_NW = _NC * _NS
_CHUNK = 128
_BB_PER_W = 4
_B_PER_W = _BB_PER_W * _CHUNK
_NCHUNK = _BB_PER_W * HIST
_FG = EMBED_DIM // 8


def _gather_body(idxt_hbm, table_hbm, p_hbm, idx_v, cbuf0, cbuf1,
                 tbuf0, tbuf1, gsem0, gsem1, wsem0, wsem1):
    wid = lax.axis_index("s") * _NC + lax.axis_index("c")

    pltpu.sync_copy(idxt_hbm.at[:, pl.ds(wid * _B_PER_W, _B_PER_W)], idx_v)

    scat_iota = lax.iota(jnp.int32, 16) * _CHUNK

    def fire(c, cbuf, gsem):
        h = lax.rem(c, HIST)
        lbb = lax.div(c, HIST)
        pltpu.async_copy(
            table_hbm.at[idx_v.at[h, pl.ds(lbb * _CHUNK, _CHUNK)]],
            cbuf, gsem)

    def drain_gather(cbuf, gsem):
        pltpu.make_async_copy(table_hbm.at[pl.ds(0, _CHUNK)], cbuf, gsem).wait()

    def transpose(cbuf, tbuf):
        def body_bi(bi, _):
            for half in range(2):
                vals = cbuf[bi, pl.ds(half * 16, 16)]
                plsc.store_scatter(tbuf, [scat_iota + (half * 2048 + bi)], vals)
            return 0
        lax.fori_loop(0, _CHUNK, body_bi, 0)

    def start_write(c, tbuf, wsem):
        h = lax.rem(c, HIST)
        wbb = wid * _BB_PER_W + lax.div(c, HIST)
        for fg in range(_FG):
            pltpu.async_copy(tbuf.at[pl.ds(fg * 1024, 1024)],
                             p_hbm.at[h, fg, wbb], wsem)

    def wait_write(tbuf, wsem):
        for fg in range(_FG):
            pltpu.make_async_copy(p_hbm.at[0, 0, 0],
                                  tbuf.at[pl.ds(0, 1024)], wsem).wait()

    fire(0, cbuf0, gsem0)
    fire(1, cbuf1, gsem1)

    def chunk_step(c, cbuf, tbuf, gsem, wsem, t):
        drain_gather(cbuf, gsem)

        @pl.when(t > 0)
        def _():
            wait_write(tbuf, wsem)
        transpose(cbuf, tbuf)

        @pl.when(c + 2 < _NCHUNK)
        def _():
            fire(c + 2, cbuf, gsem)
        start_write(c, tbuf, wsem)

    def outer(t, _):
        chunk_step(2 * t, cbuf0, tbuf0, gsem0, wsem0, t)
        chunk_step(2 * t + 1, cbuf1, tbuf1, gsem1, wsem1, t)
        return 0

    lax.fori_loop(0, _NCHUNK // 2, outer, 0)
    wait_write(tbuf0, wsem0)
    wait_write(tbuf1, wsem1)


_sc_gather = pl.kernel(
    _gather_body,
    mesh=plsc.VectorSubcoreMesh(core_axis_name="c", subcore_axis_name="s"),
    out_type=jax.ShapeDtypeStruct((HIST, _FG, BATCH // _CHUNK, 8 * _CHUNK),
                                  jnp.float32),
    scratch_types=[
        pltpu.VMEM((HIST, _B_PER_W), jnp.int32),
        pltpu.VMEM((_CHUNK, EMBED_DIM), jnp.float32),
        pltpu.VMEM((_CHUNK, EMBED_DIM), jnp.float32),
        pltpu.VMEM((_FG * 8 * _CHUNK,), jnp.float32),
        pltpu.VMEM((_FG * 8 * _CHUNK,), jnp.float32),
        pltpu.SemaphoreType.DMA,
        pltpu.SemaphoreType.DMA,
        pltpu.SemaphoreType.DMA,
        pltpu.SemaphoreType.DMA,
    ],
    compiler_params=pltpu.CompilerParams(use_tc_tiling_on_sc=False,
                                         needs_layout_passes=False),
)


def kernel(indices, embeddings):
    p = _sc_gather(indices.T, embeddings)
    p5 = p.reshape(HIST, _FG, BATCH // _CHUNK, 8, _CHUNK)
    return p5.transpose(2, 4, 0, 1, 3).reshape(BATCH, HIST, EMBED_DIM)
